# Initial kernel scaffold; baseline (speedup 1.0000x reference)
import functools

import jax
import jax.numpy as jnp
from jax.experimental import pallas as pl
from jax.experimental.pallas import tpu as pltpu

_SMOOTHING = 0.1
_CONFIDENCE = 1.0 - _SMOOTHING
_IGNORE_INDEX = -100


def _loss_body(nblocks, num_classes, block_c,
               pred_ref, tgt_ref, out_ref, m_ref, s_ref, sx_ref, g_ref):
    j = pl.program_id(0)
    n = pred_ref.shape[0]

    @pl.when(j == 0)
    def _init():
        m_ref[...] = jnp.full((n, 1), -jnp.inf, jnp.float32)
        s_ref[...] = jnp.zeros((n, 1), jnp.float32)
        sx_ref[...] = jnp.zeros((n, 1), jnp.float32)
        g_ref[...] = jnp.zeros((n, 1), jnp.float32)

    x = pred_ref[...]
    cols = j * block_c + jax.lax.broadcasted_iota(jnp.int32, (1, block_c), 1)
    valid = cols < num_classes
    xm = jnp.where(valid, x, -jnp.inf)

    m_prev = m_ref[...]
    m_new = jnp.maximum(m_prev, jnp.max(xm, axis=1, keepdims=True))
    alpha = jnp.exp(m_prev - m_new)
    s_ref[...] = s_ref[...] * alpha + jnp.sum(
        jnp.exp(xm - m_new), axis=1, keepdims=True)
    m_ref[...] = m_new
    sx_ref[...] = sx_ref[...] + jnp.sum(
        jnp.where(valid, x, 0.0), axis=1, keepdims=True)
    tmatch = cols == tgt_ref[...]
    g_ref[...] = g_ref[...] + jnp.sum(
        jnp.where(tmatch, x, 0.0), axis=1, keepdims=True)

    @pl.when(j == nblocks - 1)
    def _finish():
        lse = m_ref[...] + jnp.log(s_ref[...])
        sum_logp = sx_ref[...] - num_classes * lse
        logp_t = g_ref[...] - lse
        eps = _SMOOTHING / (num_classes - 1)
        row_loss = -eps * sum_logp - (_CONFIDENCE - eps) * logp_t
        maskf = (tgt_ref[...] != _IGNORE_INDEX).astype(jnp.float32)
        out_ref[0, 0] = jnp.sum(row_loss * maskf) / jnp.sum(maskf)


def kernel(pred, target):
    n, num_classes = pred.shape
    block_c = 2048
    nblocks = pl.cdiv(num_classes, block_c)
    tgt2 = target.reshape(n, 1)

    out = pl.pallas_call(
        functools.partial(_loss_body, nblocks, num_classes, block_c),
        grid=(nblocks,),
        in_specs=[
            pl.BlockSpec((n, block_c), lambda j: (0, j)),
            pl.BlockSpec((n, 1), lambda j: (0, 0)),
        ],
        out_specs=pl.BlockSpec((1, 1), lambda j: (0, 0)),
        out_shape=jax.ShapeDtypeStruct((1, 1), jnp.float32),
        scratch_shapes=[pltpu.VMEM((n, 1), jnp.float32)] * 4,
    )(pred, tgt2)
    return out[0, 0]


# speedup vs baseline: 2.6623x; 2.6623x over previous
import functools

import jax
import jax.numpy as jnp
from jax.experimental import pallas as pl
from jax.experimental.pallas import tpu as pltpu

_SMOOTHING = 0.1
_CONFIDENCE = 1.0 - _SMOOTHING
_IGNORE_INDEX = -100


def _loss_body(nblocks, num_classes, block_c,
               pred_ref, tgt_ref, out_ref, m_ref, s_ref, sx_ref, g_ref):
    j = pl.program_id(0)
    n = pred_ref.shape[0]

    @pl.when(j == 0)
    def _init():
        m_ref[...] = jnp.full((n, 1), -jnp.inf, jnp.float32)
        s_ref[...] = jnp.zeros((n, 1), jnp.float32)
        sx_ref[...] = jnp.zeros((n, 1), jnp.float32)
        g_ref[...] = jnp.zeros((n, 1), jnp.float32)

    x = pred_ref[...]
    cols = j * block_c + jax.lax.broadcasted_iota(jnp.int32, (1, block_c), 1)
    valid = cols < num_classes
    xm = jnp.where(valid, x, -jnp.inf)

    m_prev = m_ref[...]
    m_new = jnp.maximum(m_prev, jnp.max(xm, axis=1, keepdims=True))
    alpha = jnp.exp(m_prev - m_new)
    s_ref[...] = s_ref[...] * alpha + jnp.sum(
        jnp.exp(xm - m_new), axis=1, keepdims=True)
    m_ref[...] = m_new
    sx_ref[...] = sx_ref[...] + jnp.sum(
        jnp.where(valid, x, 0.0), axis=1, keepdims=True)
    tmatch = cols == tgt_ref[...]
    g_ref[...] = g_ref[...] + jnp.sum(
        jnp.where(tmatch, x, 0.0), axis=1, keepdims=True)

    @pl.when(j == nblocks - 1)
    def _finish():
        lse = m_ref[...] + jnp.log(s_ref[...])
        sum_logp = sx_ref[...] - num_classes * lse
        logp_t = g_ref[...] - lse
        eps = _SMOOTHING / (num_classes - 1)
        row_loss = -eps * sum_logp - (_CONFIDENCE - eps) * logp_t
        maskf = (tgt_ref[...] != _IGNORE_INDEX).astype(jnp.float32)
        loss = jnp.sum(row_loss * maskf) / jnp.sum(maskf)
        out_ref[...] = loss.reshape(1, 1)


def kernel(pred, target):
    n, num_classes = pred.shape
    block_c = 2048
    nblocks = pl.cdiv(num_classes, block_c)
    tgt2 = target.reshape(n, 1)

    out = pl.pallas_call(
        functools.partial(_loss_body, nblocks, num_classes, block_c),
        grid=(nblocks,),
        in_specs=[
            pl.BlockSpec((n, block_c), lambda j: (0, j)),
            pl.BlockSpec((n, 1), lambda j: (0, 0)),
        ],
        out_specs=pl.BlockSpec((1, 1), lambda j: (0, 0)),
        out_shape=jax.ShapeDtypeStruct((1, 1), jnp.float32),
        scratch_shapes=[pltpu.VMEM((n, 1), jnp.float32)] * 4,
    )(pred, tgt2)
    return out[0, 0]
